# baseline (speedup 1.0000x reference)
import functools

import jax
import jax.numpy as jnp
from jax import lax
from jax.experimental import pallas as pl
from jax.experimental.pallas import tpu as pltpu
from jax.experimental.pallas import tpu_sc as plsc

NUM_GRAPHS = 128
N_NODES = 100000
N_EDGES = 1600000
D_NODE = 128
D_EDGE = 16
D_GLOBAL = 64
D_OUT = 128

CH = 128
BLK = 8
NW = 32

N_CHUNKS_N = N_NODES // CH
N_TAIL = N_NODES - N_CHUNKS_N * CH
NBLK_N = N_CHUNKS_N // BLK
NTAIL_CH_N = N_CHUNKS_N - NBLK_N * BLK
N_IDROWS_N = (NBLK_N + 1) * BLK

N_CHUNKS_E = N_EDGES // CH
NBLK_E = N_CHUNKS_E // BLK
NTAIL_CH_E = N_CHUNKS_E - NBLK_E * BLK
N_IDROWS_E = (NBLK_E + 1) * BLK

NB_N, EX_N = NBLK_N // NW, NBLK_N % NW
NB_E, EX_E = NBLK_E // NW, NBLK_E % NW


def _sc_body(nfeat, efeat, nids2d, ntail_ids, eids2d,
             out_n, out_e,
             eids_v, efeat_v, nids_v, nfeat_v, ntail_ids_v, ntail_feat_v,
             zb_n, zb_e, acc_n, acc_e, sem):
    cc = lax.axis_index("c")
    sid = lax.axis_index("s")
    w = cc * 16 + sid

    for r in range(8):
        for k in range(D_NODE // 16):
            zb_n[r, pl.ds(k * 16, 16)] = jnp.zeros((16,), jnp.float32)
        zb_e[r, :] = jnp.zeros((16,), jnp.float32)
    pltpu.sync_copy(zb_n, acc_n.at[pl.ds(sid * 8, 8)])
    pltpu.sync_copy(zb_e, acc_e.at[pl.ds(sid * 8, 8)])
    plsc.subcore_barrier()

    nb0 = w * NB_N + jnp.minimum(w, EX_N)
    nbcnt = NB_N + jnp.where(w < EX_N, 1, 0)

    def node_block(b, _):
        chunk0 = (nb0 + b) * BLK
        pltpu.sync_copy(nids2d.at[pl.ds(chunk0, BLK)], nids_v)
        for j in range(BLK):
            pltpu.sync_copy(nfeat.at[pl.ds((chunk0 + j) * CH, CH)], nfeat_v)
            pltpu.sync_copy(nfeat_v, acc_n.at[nids_v.at[j]], add=True)
        return 0

    lax.fori_loop(0, nbcnt, node_block, 0)

    @pl.when(w == NW - 2)
    def _():
        chunk0 = NBLK_N * BLK
        pltpu.sync_copy(nids2d.at[pl.ds(chunk0, BLK)], nids_v)
        for j in range(NTAIL_CH_N):
            pltpu.sync_copy(nfeat.at[pl.ds((chunk0 + j) * CH, CH)], nfeat_v)
            pltpu.sync_copy(nfeat_v, acc_n.at[nids_v.at[j]], add=True)

    @pl.when(w == NW - 1)
    def _():
        pltpu.sync_copy(ntail_ids, ntail_ids_v)
        pltpu.sync_copy(nfeat.at[pl.ds(N_CHUNKS_N * CH, N_TAIL)], ntail_feat_v)
        pltpu.sync_copy(ntail_feat_v, acc_n.at[ntail_ids_v], add=True)

    eb0 = w * NB_E + jnp.minimum(w, EX_E)
    ebcnt = NB_E + jnp.where(w < EX_E, 1, 0)

    def edge_block(b, _):
        chunk0 = (eb0 + b) * BLK
        pltpu.sync_copy(eids2d.at[pl.ds(chunk0, BLK)], eids_v)
        pltpu.sync_copy(efeat.at[pl.ds(chunk0 * CH, BLK * CH)], efeat_v)
        descs = [
            pltpu.async_copy(efeat_v.at[pl.ds(j * CH, CH)],
                             acc_e.at[eids_v.at[j]], sem, add=True)
            for j in range(BLK)
        ]
        for d in descs:
            d.wait()
        return 0

    lax.fori_loop(0, ebcnt, edge_block, 0)

    @pl.when(w == NW - 1)
    def _():
        chunk0 = NBLK_E * BLK
        pltpu.sync_copy(eids2d.at[pl.ds(chunk0, BLK)], eids_v)
        pltpu.sync_copy(efeat.at[pl.ds(chunk0 * CH, NTAIL_CH_E * CH)],
                        efeat_v.at[pl.ds(0, NTAIL_CH_E * CH)])
        descs = [
            pltpu.async_copy(efeat_v.at[pl.ds(j * CH, CH)],
                             acc_e.at[eids_v.at[j]], sem, add=True)
            for j in range(NTAIL_CH_E)
        ]
        for d in descs:
            d.wait()

    plsc.subcore_barrier()

    @pl.when(sid == 0)
    def _():
        pltpu.sync_copy(acc_n, out_n.at[cc])
        pltpu.sync_copy(acc_e, out_e.at[cc])


@functools.partial(
    pl.kernel,
    out_type=(
        jax.ShapeDtypeStruct((2, NUM_GRAPHS, D_NODE), jnp.float32),
        jax.ShapeDtypeStruct((2, NUM_GRAPHS, D_EDGE), jnp.float32),
    ),
    mesh=plsc.VectorSubcoreMesh(core_axis_name="c", subcore_axis_name="s"),
    scratch_types=[
        pltpu.VMEM((BLK, CH), jnp.int32),
        pltpu.VMEM((BLK * CH, D_EDGE), jnp.float32),
        pltpu.VMEM((BLK, CH), jnp.int32),
        pltpu.VMEM((CH, D_NODE), jnp.float32),
        pltpu.VMEM((N_TAIL,), jnp.int32),
        pltpu.VMEM((N_TAIL, D_NODE), jnp.float32),
        pltpu.VMEM((8, D_NODE), jnp.float32),
        pltpu.VMEM((8, D_EDGE), jnp.float32),
        pltpu.VMEM_SHARED((NUM_GRAPHS, D_NODE), jnp.float32),
        pltpu.VMEM_SHARED((NUM_GRAPHS, D_EDGE), jnp.float32),
        pltpu.SemaphoreType.DMA,
    ],
    compiler_params=pltpu.CompilerParams(use_tc_tiling_on_sc=False),
)
def _sc_segsums(nfeat, efeat, nids2d, ntail_ids, eids2d,
                out_n, out_e, *scratch):
    _sc_body(nfeat, efeat, nids2d, ntail_ids, eids2d, out_n, out_e, *scratch)


def _final_body(pn_ref, pe_ref, g_ref, wn_ref, we_ref, wg_ref, b_ref, out_ref):
    agg_n = pn_ref[0] + pn_ref[1]
    agg_e = pe_ref[0] + pe_ref[1]
    acc = jax.lax.dot_general(
        agg_n, wn_ref[...], (((1,), (1,)), ((), ())),
        preferred_element_type=jnp.float32)
    acc += jax.lax.dot_general(
        agg_e, we_ref[...], (((1,), (1,)), ((), ())),
        preferred_element_type=jnp.float32)
    acc += jax.lax.dot_general(
        g_ref[...], wg_ref[...], (((1,), (1,)), ((), ())),
        preferred_element_type=jnp.float32)
    out_ref[...] = acc + b_ref[...]


def kernel(node_features, edge_features, global_features, node_graph_ids,
           edge_graph_ids, W_node, W_edges, W_global, bias):
    nids2d = jnp.pad(
        node_graph_ids[:N_CHUNKS_N * CH],
        (0, N_IDROWS_N * CH - N_CHUNKS_N * CH)).reshape(N_IDROWS_N, CH)
    ntail_ids = node_graph_ids[N_CHUNKS_N * CH:]
    eids2d = jnp.pad(
        edge_graph_ids,
        (0, N_IDROWS_E * CH - N_EDGES)).reshape(N_IDROWS_E, CH)
    part_n, part_e = _sc_segsums(node_features, edge_features, nids2d,
                                 ntail_ids, eids2d)
    return pl.pallas_call(
        _final_body,
        out_shape=jax.ShapeDtypeStruct((NUM_GRAPHS, D_OUT), jnp.float32),
    )(part_n, part_e, global_features, W_node, W_edges, W_global,
      bias.reshape(1, D_OUT))


# speedup vs baseline: 3.5314x; 3.5314x over previous
import functools

import jax
import jax.numpy as jnp
from jax import lax
from jax.experimental import pallas as pl
from jax.experimental.pallas import tpu as pltpu
from jax.experimental.pallas import tpu_sc as plsc

NUM_GRAPHS = 128
N_NODES = 100000
N_EDGES = 1600000
D_NODE = 128
D_EDGE = 16
D_GLOBAL = 64
D_OUT = 128

CH = 128
BLK = 8
NW = 32

N_CHUNKS_N = N_NODES // CH
N_TAIL = N_NODES - N_CHUNKS_N * CH
NBLK_N = N_CHUNKS_N // BLK
NTAIL_CH_N = N_CHUNKS_N - NBLK_N * BLK
N_IDROWS_N = (NBLK_N + 1) * BLK

N_CHUNKS_E = N_EDGES // CH
NBLK_E = N_CHUNKS_E // BLK
NTAIL_CH_E = N_CHUNKS_E - NBLK_E * BLK
N_IDROWS_E = (NBLK_E + 1) * BLK

NB_N, EX_N = NBLK_N // NW, NBLK_N % NW
NB_E, EX_E = NBLK_E // NW, NBLK_E % NW

HCH = 64
EGROUPS = BLK * CH // 16


def _sc_body(nfeat, eT, nids2d, ntail_ids, eids2d,
             out_n, out_e3,
             eids_v, efeat_v, nids_all, nfeat_v, ntail_ids_v,
             acc_n, acc_e3, sem_a, sem_b):
    cc = lax.axis_index("c")
    sid = lax.axis_index("s")
    w = cc * 16 + sid
    zvec = jnp.zeros((16,), jnp.float32)
    lane = lax.broadcasted_iota(jnp.int32, (16,), 0)

    def zrow(i, _):
        for k in range(D_NODE // 16):
            acc_n[i, pl.ds(k * 16, 16)] = zvec
        for k in range(16):
            acc_e3[i, pl.ds(k * 16, 16)] = zvec
        return 0

    lax.fori_loop(0, NUM_GRAPHS, zrow, 0)

    eb0 = w * NB_E + jnp.minimum(w, EX_E)
    ebcnt = NB_E + jnp.where(w < EX_E, 1, 0)

    def e_flush(cur, accs):
        @pl.when(cur >= 0)
        def _():
            for r in range(16):
                acc_e3[cur, pl.ds(r * 16, 16)] += accs[r]

    def e_stage(b, slot, sem):
        chunk0 = (eb0 + b) * BLK
        pltpu.async_copy(eids2d.at[pl.ds(chunk0, BLK)], eids_v.at[slot], sem)
        pltpu.async_copy(eT.at[:, pl.ds(chunk0 * CH, BLK * CH)],
                         efeat_v.at[slot], sem)

    def e_wait(b, slot, sem):
        chunk0 = (eb0 + b) * BLK
        pltpu.make_async_copy(eids2d.at[pl.ds(chunk0, BLK)],
                              eids_v.at[slot], sem).wait()
        pltpu.make_async_copy(eT.at[:, pl.ds(chunk0 * CH, BLK * CH)],
                              efeat_v.at[slot], sem).wait()

    def e_comp(slot, ngroups, carry):
        def grp(g, c):
            cur, accs = c[0], c[1:]
            j = g // 8
            col = (g - j * 8) * 16
            idvec = eids_v[slot, j, pl.ds(col, 16)]
            gid0 = idvec[0]
            gidL = idvec[15]
            gcol = g * 16
            uniform = gid0 == gidL
            is_new = jnp.logical_and(uniform, gid0 != cur)
            vs = [efeat_v[slot, r, pl.ds(gcol, 16)] for r in range(16)]

            @pl.when(jnp.logical_and(is_new, cur >= 0))
            def _():
                for r in range(16):
                    acc_e3[cur, pl.ds(r * 16, 16)] += accs[r]

            @pl.when(jnp.logical_not(uniform))
            def _():
                for r in range(16):
                    plsc.addupdate_scatter(acc_e3, [idvec, lane + r * 16],
                                           vs[r])

            accs = tuple(
                jnp.where(uniform,
                          jnp.where(is_new, zvec, accs[r]) + vs[r],
                          accs[r])
                for r in range(16))
            cur = jnp.where(uniform, gid0, cur)
            return (cur,) + accs

        return lax.fori_loop(0, ngroups, grp, carry)

    carry = (jnp.int32(-1),) + (zvec,) * 16
    e_stage(0, 0, sem_a)

    def e_pair(i, c):
        b0 = 2 * i
        b1 = b0 + 1
        e_wait(b0, 0, sem_a)
        e_stage(b1, 1, sem_b)
        c = e_comp(0, EGROUPS, c)
        e_wait(b1, 1, sem_b)

        @pl.when(b1 + 1 < ebcnt)
        def _():
            e_stage(b1 + 1, 0, sem_a)

        return e_comp(1, EGROUPS, c)

    carry = lax.fori_loop(0, ebcnt // 2, e_pair, carry)
    e_flush(carry[0], carry[1:])

    @pl.when(ebcnt % 2 == 1)
    def _():
        b = ebcnt - 1
        e_wait(b, 0, sem_a)
        c2 = e_comp(0, EGROUPS, (jnp.int32(-1),) + (zvec,) * 16)
        e_flush(c2[0], c2[1:])

    @pl.when(w == NW - 1)
    def _():
        chunk0 = NBLK_E * BLK
        pltpu.sync_copy(eids2d.at[pl.ds(chunk0, BLK)], eids_v.at[0])
        pltpu.sync_copy(eT.at[:, pl.ds(chunk0 * CH, NTAIL_CH_E * CH)],
                        efeat_v.at[0, :, pl.ds(0, NTAIL_CH_E * CH)])
        c2 = e_comp(0, NTAIL_CH_E * CH // 16,
                    (jnp.int32(-1),) + (zvec,) * 16)
        e_flush(c2[0], c2[1:])

    nb0 = w * NB_N + jnp.minimum(w, EX_N)
    nbcnt = NB_N + jnp.where(w < EX_N, 1, 0)
    pltpu.sync_copy(nids2d.at[pl.ds(nb0 * BLK, 32)], nids_all)

    def n_flush(cur, accs):
        @pl.when(cur >= 0)
        def _():
            for k in range(8):
                acc_n[cur, pl.ds(k * 16, 16)] += accs[k]

    def n_stage(h, slot, sem):
        row0 = nb0 * BLK * CH + h * HCH
        pltpu.async_copy(nfeat.at[pl.ds(row0, HCH)], nfeat_v.at[slot], sem)

    def n_wait(h, slot, sem):
        row0 = nb0 * BLK * CH + h * HCH
        pltpu.make_async_copy(nfeat.at[pl.ds(row0, HCH)],
                              nfeat_v.at[slot], sem).wait()

    def n_comp(slot, idrow, roff, carry):
        def grp(gr, c):
            cur, accs = c[0], c[1:]
            r0 = gr * 16
            nidvec = nids_all[idrow, pl.ds(roff + r0, 16)]
            gid0 = nidvec[0]
            gidL = nidvec[15]
            uniform = gid0 == gidL
            is_new = jnp.logical_and(uniform, gid0 != cur)

            @pl.when(jnp.logical_and(is_new, cur >= 0))
            def _():
                for k in range(8):
                    acc_n[cur, pl.ds(k * 16, 16)] += accs[k]

            @pl.when(jnp.logical_not(uniform))
            def _():
                for rr in range(16):
                    rid = jnp.broadcast_to(nidvec[rr], (16,))
                    for k in range(8):
                        plsc.addupdate_scatter(
                            acc_n, [rid, lane + k * 16],
                            nfeat_v[slot, r0 + rr, pl.ds(k * 16, 16)])

            tmp = tuple(jnp.where(is_new, zvec, accs[k]) for k in range(8))
            for rr in range(16):
                tmp = tuple(tmp[k] + nfeat_v[slot, r0 + rr,
                                             pl.ds(k * 16, 16)]
                            for k in range(8))
            accs = tuple(jnp.where(uniform, tmp[k], accs[k])
                         for k in range(8))
            cur = jnp.where(uniform, gid0, cur)
            return (cur,) + accs

        return lax.fori_loop(0, 4, grp, carry)

    ncarry = (jnp.int32(-1),) + (zvec,) * 8
    nhalves = nbcnt * (BLK * CH // HCH)
    n_stage(0, 0, sem_a)

    def n_pair(i, c):
        h0 = 2 * i
        h1 = h0 + 1
        n_wait(h0, 0, sem_a)
        n_stage(h1, 1, sem_b)
        c = n_comp(0, h0 // 2, (h0 % 2) * HCH, c)
        n_wait(h1, 1, sem_b)

        @pl.when(h1 + 1 < nhalves)
        def _():
            n_stage(h1 + 1, 0, sem_a)

        return n_comp(1, h1 // 2, (h1 % 2) * HCH, c)

    ncarry = lax.fori_loop(0, nhalves // 2, n_pair, ncarry)
    n_flush(ncarry[0], ncarry[1:])

    @pl.when(w == NW - 2)
    def _():
        pltpu.sync_copy(nids2d.at[pl.ds(NBLK_N * BLK, BLK)],
                        nids_all.at[pl.ds(0, BLK)])

        def one(j, c2):
            row0 = NBLK_N * BLK * CH + j * HCH
            pltpu.sync_copy(nfeat.at[pl.ds(row0, HCH)], nfeat_v.at[0])
            return n_comp(0, j // 2, (j % 2) * HCH, c2)

        c2 = lax.fori_loop(0, NTAIL_CH_N * 2, one,
                           (jnp.int32(-1),) + (zvec,) * 8)
        n_flush(c2[0], c2[1:])

    @pl.when(w == NW - 1)
    def _():
        pltpu.sync_copy(ntail_ids, ntail_ids_v)
        pltpu.sync_copy(nfeat.at[pl.ds(N_CHUNKS_N * CH, N_TAIL)],
                        nfeat_v.at[0, pl.ds(0, N_TAIL)])
        for g in range(N_TAIL // 16):
            idvec = ntail_ids_v[pl.ds(g * 16, 16)]
            for rr in range(16):
                rid = jnp.broadcast_to(idvec[rr], (16,))
                for k in range(8):
                    plsc.addupdate_scatter(
                        acc_n, [rid, lane + k * 16],
                        nfeat_v[0, g * 16 + rr, pl.ds(k * 16, 16)])

    pltpu.sync_copy(acc_n, out_n.at[w])
    pltpu.sync_copy(acc_e3, out_e3.at[w])


@functools.partial(
    pl.kernel,
    out_type=(
        jax.ShapeDtypeStruct((NW, NUM_GRAPHS, D_NODE), jnp.float32),
        jax.ShapeDtypeStruct((NW, NUM_GRAPHS, 16 * D_EDGE), jnp.float32),
    ),
    mesh=plsc.VectorSubcoreMesh(core_axis_name="c", subcore_axis_name="s"),
    scratch_types=[
        pltpu.VMEM((2, BLK, CH), jnp.int32),
        pltpu.VMEM((2, D_EDGE, BLK * CH), jnp.float32),
        pltpu.VMEM((32, CH), jnp.int32),
        pltpu.VMEM((2, HCH, D_NODE), jnp.float32),
        pltpu.VMEM((N_TAIL,), jnp.int32),
        pltpu.VMEM((NUM_GRAPHS, D_NODE), jnp.float32),
        pltpu.VMEM((NUM_GRAPHS, 16 * D_EDGE), jnp.float32),
        pltpu.SemaphoreType.DMA,
        pltpu.SemaphoreType.DMA,
    ],
    compiler_params=pltpu.CompilerParams(needs_layout_passes=False),
)
def _sc_segsums(nfeat, eT, nids2d, ntail_ids, eids2d, out_n, out_e3,
                *scratch):
    _sc_body(nfeat, eT, nids2d, ntail_ids, eids2d, out_n, out_e3, *scratch)


def _final_body(pn_ref, pe_ref, fold_ref, g_ref, wn_ref, we_ref, wg_ref,
                b_ref, out_ref):
    agg_n = jnp.sum(pn_ref[...], axis=0)
    pe = jnp.sum(pe_ref[...], axis=0)
    agg_e = jax.lax.dot_general(
        pe, fold_ref[...], (((1,), (0,)), ((), ())),
        precision=jax.lax.Precision.HIGHEST,
        preferred_element_type=jnp.float32)
    acc = jax.lax.dot_general(
        agg_n, wn_ref[...], (((1,), (1,)), ((), ())),
        preferred_element_type=jnp.float32)
    acc += jax.lax.dot_general(
        agg_e, we_ref[...], (((1,), (1,)), ((), ())),
        preferred_element_type=jnp.float32)
    acc += jax.lax.dot_general(
        g_ref[...], wg_ref[...], (((1,), (1,)), ((), ())),
        preferred_element_type=jnp.float32)
    out_ref[...] = acc + b_ref[...]


def kernel(node_features, edge_features, global_features, node_graph_ids,
           edge_graph_ids, W_node, W_edges, W_global, bias):
    nids2d = jnp.pad(
        node_graph_ids[:N_CHUNKS_N * CH],
        (0, N_IDROWS_N * CH - N_CHUNKS_N * CH)).reshape(N_IDROWS_N, CH)
    ntail_ids = node_graph_ids[N_CHUNKS_N * CH:]
    eids2d = jnp.pad(
        edge_graph_ids,
        (0, N_IDROWS_E * CH - N_EDGES)).reshape(N_IDROWS_E, CH)
    part_n, part_e3 = _sc_segsums(node_features, edge_features.T, nids2d,
                                  ntail_ids, eids2d)
    fold = (jnp.arange(16 * D_EDGE)[:, None] // 16
            == jnp.arange(D_EDGE)[None, :]).astype(jnp.float32)
    return pl.pallas_call(
        _final_body,
        out_shape=jax.ShapeDtypeStruct((NUM_GRAPHS, D_OUT), jnp.float32),
    )(part_n, part_e3, fold, global_features, W_node, W_edges, W_global,
      bias.reshape(1, D_OUT))


# speedup vs baseline: 4.9903x; 1.4131x over previous
import functools

import jax
import jax.numpy as jnp
from jax import lax
from jax.experimental import pallas as pl
from jax.experimental.pallas import tpu as pltpu
from jax.experimental.pallas import tpu_sc as plsc

NUM_GRAPHS = 128
N_NODES = 100000
N_EDGES = 1600000
D_NODE = 128
D_EDGE = 16
D_GLOBAL = 64
D_OUT = 128

CH = 128
BLK = 8
NW = 32

N_CHUNKS_N = N_NODES // CH
N_TAIL = N_NODES - N_CHUNKS_N * CH
NBLK_N = N_CHUNKS_N // BLK
NTAIL_CH_N = N_CHUNKS_N - NBLK_N * BLK
N_IDROWS_N = (NBLK_N + 1) * BLK

N_CHUNKS_E = N_EDGES // CH
NBLK_E = N_CHUNKS_E // BLK
NTAIL_CH_E = N_CHUNKS_E - NBLK_E * BLK
N_IDROWS_E = (NBLK_E + 1) * BLK

NB_N, EX_N = NBLK_N // NW, NBLK_N % NW
NB_E, EX_E = NBLK_E // NW, NBLK_E % NW

HCH = 64
EGROUPS = BLK * CH // 16


def _sc_body(nfeat, eT, nids2d, ntail_ids, eids2d,
             out_n, out_e3,
             eids_v, efeat_v, nids_all, nfeat_v, ntail_ids_v,
             acc_n, acc_e3, sem_a, sem_b):
    cc = lax.axis_index("c")
    sid = lax.axis_index("s")
    w = cc * 16 + sid
    zvec = jnp.zeros((16,), jnp.float32)
    lane = lax.broadcasted_iota(jnp.int32, (16,), 0)

    def zrow(i, _):
        for k in range(D_NODE // 16):
            acc_n[i, pl.ds(k * 16, 16)] = zvec
        for k in range(16):
            acc_e3[i, pl.ds(k * 16, 16)] = zvec
        return 0

    lax.fori_loop(0, NUM_GRAPHS, zrow, 0)

    eb0 = w * NB_E + jnp.minimum(w, EX_E)
    ebcnt = NB_E + jnp.where(w < EX_E, 1, 0)

    def e_flush(cur, accs):
        @pl.when(cur >= 0)
        def _():
            for r in range(16):
                acc_e3[cur, pl.ds(r * 16, 16)] += accs[r]

    def e_stage(b, slot, sem):
        chunk0 = (eb0 + b) * BLK
        pltpu.async_copy(eids2d.at[pl.ds(chunk0, BLK)], eids_v.at[slot], sem)
        pltpu.async_copy(eT.at[:, pl.ds(chunk0 * CH, BLK * CH)],
                         efeat_v.at[slot], sem)

    def e_wait(b, slot, sem):
        chunk0 = (eb0 + b) * BLK
        pltpu.make_async_copy(eids2d.at[pl.ds(chunk0, BLK)],
                              eids_v.at[slot], sem).wait()
        pltpu.make_async_copy(eT.at[:, pl.ds(chunk0 * CH, BLK * CH)],
                              efeat_v.at[slot], sem).wait()

    def e_block(slot, nchunks):
        bid0 = eids_v[slot, 0, pl.ds(0, 16)][0]
        bidL = eids_v[slot, nchunks - 1, pl.ds(CH - 16, 16)][15]

        @pl.when(bid0 == bidL)
        def _():
            def g2(i, accs):
                accs = tuple(accs[r] + efeat_v[slot, r, pl.ds(i * 32, 16)]
                             for r in range(16))
                return tuple(accs[r] + efeat_v[slot, r,
                                               pl.ds(i * 32 + 16, 16)]
                             for r in range(16))

            accs = lax.fori_loop(0, nchunks * 4, g2, (zvec,) * 16)
            for r in range(16):
                acc_e3[bid0, pl.ds(r * 16, 16)] += accs[r]

        @pl.when(bid0 != bidL)
        def _():
            def grp(g, c):
                cur, accs = c[0], c[1:]
                j = g // 8
                col = (g - j * 8) * 16
                idvec = eids_v[slot, j, pl.ds(col, 16)]
                gid0 = idvec[0]
                gidL = idvec[15]
                gcol = g * 16
                uniform = gid0 == gidL
                is_new = jnp.logical_and(uniform, gid0 != cur)
                vs = [efeat_v[slot, r, pl.ds(gcol, 16)] for r in range(16)]

                @pl.when(jnp.logical_and(is_new, cur >= 0))
                def _():
                    for r in range(16):
                        acc_e3[cur, pl.ds(r * 16, 16)] += accs[r]

                @pl.when(jnp.logical_not(uniform))
                def _():
                    for r in range(16):
                        plsc.addupdate_scatter(acc_e3,
                                               [idvec, lane + r * 16], vs[r])

                accs = tuple(
                    jnp.where(uniform,
                              jnp.where(is_new, zvec, accs[r]) + vs[r],
                              accs[r])
                    for r in range(16))
                cur = jnp.where(uniform, gid0, cur)
                return (cur,) + accs

            c = lax.fori_loop(0, nchunks * 8, grp,
                              (jnp.int32(-1),) + (zvec,) * 16)
            e_flush(c[0], c[1:])

    e_stage(0, 0, sem_a)

    def e_pair(i, _):
        b0 = 2 * i
        b1 = b0 + 1
        e_wait(b0, 0, sem_a)
        e_stage(b1, 1, sem_b)
        e_block(0, BLK)
        e_wait(b1, 1, sem_b)

        @pl.when(b1 + 1 < ebcnt)
        def _():
            e_stage(b1 + 1, 0, sem_a)

        e_block(1, BLK)
        return 0

    lax.fori_loop(0, ebcnt // 2, e_pair, 0)

    @pl.when(ebcnt % 2 == 1)
    def _():
        e_wait(ebcnt - 1, 0, sem_a)
        e_block(0, BLK)

    @pl.when(w == NW - 1)
    def _():
        chunk0 = NBLK_E * BLK
        pltpu.sync_copy(eids2d.at[pl.ds(chunk0, BLK)], eids_v.at[0])
        pltpu.sync_copy(eT.at[:, pl.ds(chunk0 * CH, NTAIL_CH_E * CH)],
                        efeat_v.at[0, :, pl.ds(0, NTAIL_CH_E * CH)])
        e_block(0, NTAIL_CH_E)

    nb0 = w * NB_N + jnp.minimum(w, EX_N)
    nbcnt = NB_N + jnp.where(w < EX_N, 1, 0)
    pltpu.sync_copy(nids2d.at[pl.ds(nb0 * BLK, 32)], nids_all)

    def n_flush(cur, accs):
        @pl.when(cur >= 0)
        def _():
            for k in range(8):
                acc_n[cur, pl.ds(k * 16, 16)] += accs[k]

    def n_stage(h, slot, sem):
        row0 = nb0 * BLK * CH + h * HCH
        pltpu.async_copy(nfeat.at[pl.ds(row0, HCH)], nfeat_v.at[slot], sem)

    def n_wait(h, slot, sem):
        row0 = nb0 * BLK * CH + h * HCH
        pltpu.make_async_copy(nfeat.at[pl.ds(row0, HCH)],
                              nfeat_v.at[slot], sem).wait()

    def n_half(slot, idrow, roff):
        hid0 = nids_all[idrow, pl.ds(roff, 16)][0]
        hidL = nids_all[idrow, pl.ds(roff + HCH - 16, 16)][15]

        @pl.when(hid0 == hidL)
        def _():
            def r2(i, accs):
                accs = tuple(accs[k] + nfeat_v[slot, 2 * i,
                                               pl.ds(k * 16, 16)]
                             for k in range(8))
                return tuple(accs[k] + nfeat_v[slot, 2 * i + 1,
                                               pl.ds(k * 16, 16)]
                             for k in range(8))

            accs = lax.fori_loop(0, HCH // 2, r2, (zvec,) * 8)
            for k in range(8):
                acc_n[hid0, pl.ds(k * 16, 16)] += accs[k]

        @pl.when(hid0 != hidL)
        def _():
            def grp(gr, c):
                cur, accs = c[0], c[1:]
                r0 = gr * 16
                nidvec = nids_all[idrow, pl.ds(roff + r0, 16)]
                gid0 = nidvec[0]
                gidL = nidvec[15]
                uniform = gid0 == gidL
                is_new = jnp.logical_and(uniform, gid0 != cur)

                @pl.when(jnp.logical_and(is_new, cur >= 0))
                def _():
                    for k in range(8):
                        acc_n[cur, pl.ds(k * 16, 16)] += accs[k]

                @pl.when(jnp.logical_not(uniform))
                def _():
                    for rr in range(16):
                        rid = jnp.broadcast_to(nidvec[rr], (16,))
                        for k in range(8):
                            plsc.addupdate_scatter(
                                acc_n, [rid, lane + k * 16],
                                nfeat_v[slot, r0 + rr, pl.ds(k * 16, 16)])

                tmp = tuple(jnp.where(is_new, zvec, accs[k])
                            for k in range(8))
                for rr in range(16):
                    tmp = tuple(tmp[k] + nfeat_v[slot, r0 + rr,
                                                 pl.ds(k * 16, 16)]
                                for k in range(8))
                accs = tuple(jnp.where(uniform, tmp[k], accs[k])
                             for k in range(8))
                cur = jnp.where(uniform, gid0, cur)
                return (cur,) + accs

            c = lax.fori_loop(0, 4, grp, (jnp.int32(-1),) + (zvec,) * 8)
            n_flush(c[0], c[1:])

    nhalves = nbcnt * (BLK * CH // HCH)
    n_stage(0, 0, sem_a)

    def n_pair(i, _):
        h0 = 2 * i
        h1 = h0 + 1
        n_wait(h0, 0, sem_a)
        n_stage(h1, 1, sem_b)
        n_half(0, h0 // 2, (h0 % 2) * HCH)
        n_wait(h1, 1, sem_b)

        @pl.when(h1 + 1 < nhalves)
        def _():
            n_stage(h1 + 1, 0, sem_a)

        n_half(1, h1 // 2, (h1 % 2) * HCH)
        return 0

    lax.fori_loop(0, nhalves // 2, n_pair, 0)

    @pl.when(w == NW - 2)
    def _():
        pltpu.sync_copy(nids2d.at[pl.ds(NBLK_N * BLK, BLK)],
                        nids_all.at[pl.ds(0, BLK)])

        def one(j, _):
            row0 = NBLK_N * BLK * CH + j * HCH
            pltpu.sync_copy(nfeat.at[pl.ds(row0, HCH)], nfeat_v.at[0])
            n_half(0, j // 2, (j % 2) * HCH)
            return 0

        lax.fori_loop(0, NTAIL_CH_N * 2, one, 0)

    @pl.when(w == NW - 1)
    def _():
        pltpu.sync_copy(ntail_ids, ntail_ids_v)
        pltpu.sync_copy(nfeat.at[pl.ds(N_CHUNKS_N * CH, N_TAIL)],
                        nfeat_v.at[0, pl.ds(0, N_TAIL)])
        for g in range(N_TAIL // 16):
            idvec = ntail_ids_v[pl.ds(g * 16, 16)]
            for rr in range(16):
                rid = jnp.broadcast_to(idvec[rr], (16,))
                for k in range(8):
                    plsc.addupdate_scatter(
                        acc_n, [rid, lane + k * 16],
                        nfeat_v[0, g * 16 + rr, pl.ds(k * 16, 16)])

    pltpu.sync_copy(acc_n, out_n.at[w])
    pltpu.sync_copy(acc_e3, out_e3.at[w])


@functools.partial(
    pl.kernel,
    out_type=(
        jax.ShapeDtypeStruct((NW, NUM_GRAPHS, D_NODE), jnp.float32),
        jax.ShapeDtypeStruct((NW, NUM_GRAPHS, 16 * D_EDGE), jnp.float32),
    ),
    mesh=plsc.VectorSubcoreMesh(core_axis_name="c", subcore_axis_name="s"),
    scratch_types=[
        pltpu.VMEM((2, BLK, CH), jnp.int32),
        pltpu.VMEM((2, D_EDGE, BLK * CH), jnp.float32),
        pltpu.VMEM((32, CH), jnp.int32),
        pltpu.VMEM((2, HCH, D_NODE), jnp.float32),
        pltpu.VMEM((N_TAIL,), jnp.int32),
        pltpu.VMEM((NUM_GRAPHS, D_NODE), jnp.float32),
        pltpu.VMEM((NUM_GRAPHS, 16 * D_EDGE), jnp.float32),
        pltpu.SemaphoreType.DMA,
        pltpu.SemaphoreType.DMA,
    ],
    compiler_params=pltpu.CompilerParams(needs_layout_passes=False),
)
def _sc_segsums(nfeat, eT, nids2d, ntail_ids, eids2d, out_n, out_e3,
                *scratch):
    _sc_body(nfeat, eT, nids2d, ntail_ids, eids2d, out_n, out_e3, *scratch)


def _final_body(pn_ref, pe_ref, fold_ref, g_ref, wn_ref, we_ref, wg_ref,
                b_ref, out_ref):
    agg_n = jnp.sum(pn_ref[...], axis=0)
    pe = jnp.sum(pe_ref[...], axis=0)
    agg_e = jax.lax.dot_general(
        pe, fold_ref[...], (((1,), (0,)), ((), ())),
        precision=jax.lax.Precision.HIGHEST,
        preferred_element_type=jnp.float32)
    acc = jax.lax.dot_general(
        agg_n, wn_ref[...], (((1,), (1,)), ((), ())),
        preferred_element_type=jnp.float32)
    acc += jax.lax.dot_general(
        agg_e, we_ref[...], (((1,), (1,)), ((), ())),
        preferred_element_type=jnp.float32)
    acc += jax.lax.dot_general(
        g_ref[...], wg_ref[...], (((1,), (1,)), ((), ())),
        preferred_element_type=jnp.float32)
    out_ref[...] = acc + b_ref[...]


def kernel(node_features, edge_features, global_features, node_graph_ids,
           edge_graph_ids, W_node, W_edges, W_global, bias):
    nids2d = jnp.pad(
        node_graph_ids[:N_CHUNKS_N * CH],
        (0, N_IDROWS_N * CH - N_CHUNKS_N * CH)).reshape(N_IDROWS_N, CH)
    ntail_ids = node_graph_ids[N_CHUNKS_N * CH:]
    eids2d = jnp.pad(
        edge_graph_ids,
        (0, N_IDROWS_E * CH - N_EDGES)).reshape(N_IDROWS_E, CH)
    part_n, part_e3 = _sc_segsums(node_features, edge_features.T, nids2d,
                                  ntail_ids, eids2d)
    fold = (jnp.arange(16 * D_EDGE)[:, None] // 16
            == jnp.arange(D_EDGE)[None, :]).astype(jnp.float32)
    return pl.pallas_call(
        _final_body,
        out_shape=jax.ShapeDtypeStruct((NUM_GRAPHS, D_OUT), jnp.float32),
    )(part_n, part_e3, fold, global_features, W_node, W_edges, W_global,
      bias.reshape(1, D_OUT))


# speedup vs baseline: 4.9909x; 1.0001x over previous
import functools

import jax
import jax.numpy as jnp
from jax import lax
from jax.experimental import pallas as pl
from jax.experimental.pallas import tpu as pltpu
from jax.experimental.pallas import tpu_sc as plsc

NUM_GRAPHS = 128
N_NODES = 100000
N_EDGES = 1600000
D_NODE = 128
D_EDGE = 16
D_GLOBAL = 64
D_OUT = 128

CH = 128
BLK = 8
NW = 32

N_CHUNKS_N = N_NODES // CH
N_TAIL = N_NODES - N_CHUNKS_N * CH
NBLK_N = N_CHUNKS_N // BLK
NTAIL_CH_N = N_CHUNKS_N - NBLK_N * BLK
N_IDROWS_N = (NBLK_N + 1) * BLK

N_CHUNKS_E = N_EDGES // CH
NBLK_E = N_CHUNKS_E // BLK
NTAIL_CH_E = N_CHUNKS_E - NBLK_E * BLK
N_IDROWS_E = (NBLK_E + 1) * BLK

NB_N, EX_N = NBLK_N // NW, NBLK_N % NW
NB_E, EX_E = NBLK_E // NW, NBLK_E % NW

HCH = 64
EGROUPS = BLK * CH // 16


def _sc_body(nfeat, eT, nids2d, ntail_ids, eids2d,
             out_n, out_e3,
             eids_v, efeat_v, nids_all, nfeat_v, ntail_ids_v,
             acc_n, acc_e3, sem_a, sem_b):
    cc = lax.axis_index("c")
    sid = lax.axis_index("s")
    w = sid * 2 + cc
    zvec = jnp.zeros((16,), jnp.float32)
    lane = lax.broadcasted_iota(jnp.int32, (16,), 0)

    def zrow(i, _):
        for k in range(D_NODE // 16):
            acc_n[i, pl.ds(k * 16, 16)] = zvec
        for k in range(16):
            acc_e3[i, pl.ds(k * 16, 16)] = zvec
        return 0

    lax.fori_loop(0, NUM_GRAPHS, zrow, 0)

    eb0 = w * NB_E + jnp.minimum(w, EX_E)
    ebcnt = NB_E + jnp.where(w < EX_E, 1, 0)

    def e_flush(cur, accs):
        @pl.when(cur >= 0)
        def _():
            for r in range(16):
                acc_e3[cur, pl.ds(r * 16, 16)] += accs[r]

    def e_stage(b, slot, sem):
        chunk0 = (eb0 + b) * BLK
        pltpu.async_copy(eids2d.at[pl.ds(chunk0, BLK)], eids_v.at[slot], sem)
        pltpu.async_copy(eT.at[:, pl.ds(chunk0 * CH, BLK * CH)],
                         efeat_v.at[slot], sem)

    def e_wait(b, slot, sem):
        chunk0 = (eb0 + b) * BLK
        pltpu.make_async_copy(eids2d.at[pl.ds(chunk0, BLK)],
                              eids_v.at[slot], sem).wait()
        pltpu.make_async_copy(eT.at[:, pl.ds(chunk0 * CH, BLK * CH)],
                              efeat_v.at[slot], sem).wait()

    def e_block(slot, nchunks):
        bid0 = eids_v[slot, 0, pl.ds(0, 16)][0]
        bidL = eids_v[slot, nchunks - 1, pl.ds(CH - 16, 16)][15]

        @pl.when(bid0 == bidL)
        def _():
            def g4(i, accs):
                for u in range(4):
                    accs = tuple(
                        accs[r] + efeat_v[slot, r, pl.ds(i * 64 + u * 16, 16)]
                        for r in range(16))
                return accs

            accs = lax.fori_loop(0, nchunks * 2, g4, (zvec,) * 16)
            for r in range(16):
                acc_e3[bid0, pl.ds(r * 16, 16)] += accs[r]

        @pl.when(bid0 != bidL)
        def _():
            def grp(g, c):
                cur, accs = c[0], c[1:]
                j = g // 8
                col = (g - j * 8) * 16
                idvec = eids_v[slot, j, pl.ds(col, 16)]
                gid0 = idvec[0]
                gidL = idvec[15]
                gcol = g * 16
                uniform = gid0 == gidL
                is_new = jnp.logical_and(uniform, gid0 != cur)
                vs = [efeat_v[slot, r, pl.ds(gcol, 16)] for r in range(16)]

                @pl.when(jnp.logical_and(is_new, cur >= 0))
                def _():
                    for r in range(16):
                        acc_e3[cur, pl.ds(r * 16, 16)] += accs[r]

                @pl.when(jnp.logical_not(uniform))
                def _():
                    for r in range(16):
                        plsc.addupdate_scatter(acc_e3,
                                               [idvec, lane + r * 16], vs[r])

                accs = tuple(
                    jnp.where(uniform,
                              jnp.where(is_new, zvec, accs[r]) + vs[r],
                              accs[r])
                    for r in range(16))
                cur = jnp.where(uniform, gid0, cur)
                return (cur,) + accs

            c = lax.fori_loop(0, nchunks * 8, grp,
                              (jnp.int32(-1),) + (zvec,) * 16)
            e_flush(c[0], c[1:])

    e_stage(0, 0, sem_a)

    def e_pair(i, _):
        b0 = 2 * i
        b1 = b0 + 1
        e_wait(b0, 0, sem_a)
        e_stage(b1, 1, sem_b)
        e_block(0, BLK)
        e_wait(b1, 1, sem_b)

        @pl.when(b1 + 1 < ebcnt)
        def _():
            e_stage(b1 + 1, 0, sem_a)

        e_block(1, BLK)
        return 0

    lax.fori_loop(0, ebcnt // 2, e_pair, 0)

    @pl.when(ebcnt % 2 == 1)
    def _():
        e_wait(ebcnt - 1, 0, sem_a)
        e_block(0, BLK)

    @pl.when(w == NW - 1)
    def _():
        chunk0 = NBLK_E * BLK
        pltpu.sync_copy(eids2d.at[pl.ds(chunk0, NTAIL_CH_E)],
                        eids_v.at[0, pl.ds(0, NTAIL_CH_E)])
        pltpu.sync_copy(eT.at[:, pl.ds(chunk0 * CH, NTAIL_CH_E * CH)],
                        efeat_v.at[0, :, pl.ds(0, NTAIL_CH_E * CH)])
        e_block(0, NTAIL_CH_E)

    nb0 = w * NB_N + jnp.minimum(w, EX_N)
    nbcnt = NB_N + jnp.where(w < EX_N, 1, 0)
    pltpu.sync_copy(nids2d.at[pl.ds(nb0 * BLK, 32)], nids_all)

    def n_flush(cur, accs):
        @pl.when(cur >= 0)
        def _():
            for k in range(8):
                acc_n[cur, pl.ds(k * 16, 16)] += accs[k]

    def n_stage(h, slot, sem):
        row0 = nb0 * BLK * CH + h * HCH
        pltpu.async_copy(nfeat.at[pl.ds(row0, HCH)], nfeat_v.at[slot], sem)

    def n_wait(h, slot, sem):
        row0 = nb0 * BLK * CH + h * HCH
        pltpu.make_async_copy(nfeat.at[pl.ds(row0, HCH)],
                              nfeat_v.at[slot], sem).wait()

    def n_half(slot, idrow, roff):
        hid0 = nids_all[idrow, pl.ds(roff, 16)][0]
        hidL = nids_all[idrow, pl.ds(roff + HCH - 16, 16)][15]

        @pl.when(hid0 == hidL)
        def _():
            def r4(i, accs):
                for u in range(4):
                    accs = tuple(accs[k] + nfeat_v[slot, 4 * i + u,
                                                   pl.ds(k * 16, 16)]
                                 for k in range(8))
                return accs

            accs = lax.fori_loop(0, HCH // 4, r4, (zvec,) * 8)
            for k in range(8):
                acc_n[hid0, pl.ds(k * 16, 16)] += accs[k]

        @pl.when(hid0 != hidL)
        def _():
            def grp(gr, c):
                cur, accs = c[0], c[1:]
                r0 = gr * 16
                nidvec = nids_all[idrow, pl.ds(roff + r0, 16)]
                gid0 = nidvec[0]
                gidL = nidvec[15]
                uniform = gid0 == gidL
                is_new = jnp.logical_and(uniform, gid0 != cur)

                @pl.when(jnp.logical_and(is_new, cur >= 0))
                def _():
                    for k in range(8):
                        acc_n[cur, pl.ds(k * 16, 16)] += accs[k]

                @pl.when(jnp.logical_not(uniform))
                def _():
                    for rr in range(16):
                        rid = jnp.broadcast_to(nidvec[rr], (16,))
                        for k in range(8):
                            plsc.addupdate_scatter(
                                acc_n, [rid, lane + k * 16],
                                nfeat_v[slot, r0 + rr, pl.ds(k * 16, 16)])

                tmp = tuple(jnp.where(is_new, zvec, accs[k])
                            for k in range(8))
                for rr in range(16):
                    tmp = tuple(tmp[k] + nfeat_v[slot, r0 + rr,
                                                 pl.ds(k * 16, 16)]
                                for k in range(8))
                accs = tuple(jnp.where(uniform, tmp[k], accs[k])
                             for k in range(8))
                cur = jnp.where(uniform, gid0, cur)
                return (cur,) + accs

            c = lax.fori_loop(0, 4, grp, (jnp.int32(-1),) + (zvec,) * 8)
            n_flush(c[0], c[1:])

    nhalves = nbcnt * (BLK * CH // HCH)
    n_stage(0, 0, sem_a)

    def n_pair(i, _):
        h0 = 2 * i
        h1 = h0 + 1
        n_wait(h0, 0, sem_a)
        n_stage(h1, 1, sem_b)
        n_half(0, h0 // 2, (h0 % 2) * HCH)
        n_wait(h1, 1, sem_b)

        @pl.when(h1 + 1 < nhalves)
        def _():
            n_stage(h1 + 1, 0, sem_a)

        n_half(1, h1 // 2, (h1 % 2) * HCH)
        return 0

    lax.fori_loop(0, nhalves // 2, n_pair, 0)

    @pl.when(w == NW - 2)
    def _():
        pltpu.sync_copy(nids2d.at[pl.ds(NBLK_N * BLK, BLK)],
                        nids_all.at[pl.ds(0, BLK)])

        def one(j, _):
            row0 = NBLK_N * BLK * CH + j * HCH
            pltpu.sync_copy(nfeat.at[pl.ds(row0, HCH)], nfeat_v.at[0])
            n_half(0, j // 2, (j % 2) * HCH)
            return 0

        lax.fori_loop(0, NTAIL_CH_N * 2, one, 0)

    @pl.when(w == NW - 1)
    def _():
        pltpu.sync_copy(ntail_ids, ntail_ids_v)
        pltpu.sync_copy(nfeat.at[pl.ds(N_CHUNKS_N * CH, N_TAIL)],
                        nfeat_v.at[0, pl.ds(0, N_TAIL)])
        for g in range(N_TAIL // 16):
            idvec = ntail_ids_v[pl.ds(g * 16, 16)]
            for rr in range(16):
                rid = jnp.broadcast_to(idvec[rr], (16,))
                for k in range(8):
                    plsc.addupdate_scatter(
                        acc_n, [rid, lane + k * 16],
                        nfeat_v[0, g * 16 + rr, pl.ds(k * 16, 16)])

    pltpu.sync_copy(acc_n, out_n.at[w])
    pltpu.sync_copy(acc_e3, out_e3.at[w])


@functools.partial(
    pl.kernel,
    out_type=(
        jax.ShapeDtypeStruct((NW, NUM_GRAPHS, D_NODE), jnp.float32),
        jax.ShapeDtypeStruct((NW, NUM_GRAPHS, 16 * D_EDGE), jnp.float32),
    ),
    mesh=plsc.VectorSubcoreMesh(core_axis_name="c", subcore_axis_name="s"),
    scratch_types=[
        pltpu.VMEM((2, BLK, CH), jnp.int32),
        pltpu.VMEM((2, D_EDGE, BLK * CH), jnp.float32),
        pltpu.VMEM((32, CH), jnp.int32),
        pltpu.VMEM((2, HCH, D_NODE), jnp.float32),
        pltpu.VMEM((N_TAIL,), jnp.int32),
        pltpu.VMEM((NUM_GRAPHS, D_NODE), jnp.float32),
        pltpu.VMEM((NUM_GRAPHS, 16 * D_EDGE), jnp.float32),
        pltpu.SemaphoreType.DMA,
        pltpu.SemaphoreType.DMA,
    ],
    compiler_params=pltpu.CompilerParams(needs_layout_passes=False),
)
def _sc_segsums(nfeat, eT, nids2d, ntail_ids, eids2d, out_n, out_e3,
                *scratch):
    _sc_body(nfeat, eT, nids2d, ntail_ids, eids2d, out_n, out_e3, *scratch)


def _final_body(pn_ref, pe_ref, fold_ref, g_ref, wn_ref, we_ref, wg_ref,
                b_ref, out_ref):
    agg_n = jnp.sum(pn_ref[...], axis=0)
    pe = jnp.sum(pe_ref[...], axis=0)
    agg_e = jax.lax.dot_general(
        pe, fold_ref[...], (((1,), (0,)), ((), ())),
        precision=jax.lax.Precision.HIGHEST,
        preferred_element_type=jnp.float32)
    acc = jax.lax.dot_general(
        agg_n, wn_ref[...], (((1,), (1,)), ((), ())),
        preferred_element_type=jnp.float32)
    acc += jax.lax.dot_general(
        agg_e, we_ref[...], (((1,), (1,)), ((), ())),
        preferred_element_type=jnp.float32)
    acc += jax.lax.dot_general(
        g_ref[...], wg_ref[...], (((1,), (1,)), ((), ())),
        preferred_element_type=jnp.float32)
    out_ref[...] = acc + b_ref[...]


def kernel(node_features, edge_features, global_features, node_graph_ids,
           edge_graph_ids, W_node, W_edges, W_global, bias):
    nids2d = jnp.pad(
        node_graph_ids[:N_CHUNKS_N * CH],
        (0, N_IDROWS_N * CH - N_CHUNKS_N * CH)).reshape(N_IDROWS_N, CH)
    ntail_ids = node_graph_ids[N_CHUNKS_N * CH:]
    eids2d = edge_graph_ids.reshape(N_CHUNKS_E, CH)
    part_n, part_e3 = _sc_segsums(node_features, edge_features.T, nids2d,
                                  ntail_ids, eids2d)
    fold = (jnp.arange(16 * D_EDGE)[:, None] // 16
            == jnp.arange(D_EDGE)[None, :]).astype(jnp.float32)
    return pl.pallas_call(
        _final_body,
        out_shape=jax.ShapeDtypeStruct((NUM_GRAPHS, D_OUT), jnp.float32),
    )(part_n, part_e3, fold, global_features, W_node, W_edges, W_global,
      bias.reshape(1, D_OUT))


# speedup vs baseline: 7.7222x; 1.5473x over previous
import functools

import jax
import jax.numpy as jnp
from jax import lax
from jax.experimental import pallas as pl
from jax.experimental.pallas import tpu as pltpu
from jax.experimental.pallas import tpu_sc as plsc

NUM_GRAPHS = 128
N_NODES = 100000
N_EDGES = 1600000
D_NODE = 128
D_EDGE = 16
D_GLOBAL = 64
D_OUT = 128

CH = 128
BLK = 8
NW = 32

N_CHUNKS_N = N_NODES // CH
N_TAIL = N_NODES - N_CHUNKS_N * CH
NBLK_N = N_CHUNKS_N // BLK
NTAIL_CH_N = N_CHUNKS_N - NBLK_N * BLK
N_IDROWS_N = (NBLK_N + 1) * BLK

N_CHUNKS_E = N_EDGES // CH
NBLK_E = N_CHUNKS_E // BLK
NTAIL_CH_E = N_CHUNKS_E - NBLK_E * BLK
N_IDROWS_E = (NBLK_E + 1) * BLK

NB_N, EX_N = NBLK_N // NW, NBLK_N % NW
NB_E, EX_E = NBLK_E // NW, NBLK_E % NW

HCH = 64
EGROUPS = BLK * CH // 16


def _sc_body(nfeat, eT, nids2d, ntail_ids, eids2d,
             out_n, out_e3,
             eids_v, efeat_v, nids_all, nfeat_v, ntail_ids_v,
             acc_n, acc_e3, sem_a, sem_b, sem_c, sem_d):
    cc = lax.axis_index("c")
    sid = lax.axis_index("s")
    w = sid * 2 + cc
    zvec = jnp.zeros((16,), jnp.float32)
    lane = lax.broadcasted_iota(jnp.int32, (16,), 0)

    def zrow(i, _):
        for k in range(D_NODE // 16):
            acc_n[i, pl.ds(k * 16, 16)] = zvec
        for k in range(16):
            acc_e3[i, pl.ds(k * 16, 16)] = zvec
        return 0

    lax.fori_loop(0, NUM_GRAPHS, zrow, 0)

    eb0 = w * NB_E + jnp.minimum(w, EX_E)
    ebcnt = NB_E + jnp.where(w < EX_E, 1, 0)

    def e_flush(cur, accs):
        @pl.when(cur >= 0)
        def _():
            for r in range(16):
                acc_e3[cur, pl.ds(r * 16, 16)] += accs[r]

    def e_stage(b, slot, sem):
        chunk0 = (eb0 + b) * BLK
        pltpu.async_copy(eids2d.at[pl.ds(chunk0, BLK)], eids_v.at[slot], sem)
        pltpu.async_copy(eT.at[:, pl.ds(chunk0 * CH, BLK * CH)],
                         efeat_v.at[slot], sem)

    def e_wait(b, slot, sem):
        chunk0 = (eb0 + b) * BLK
        pltpu.make_async_copy(eids2d.at[pl.ds(chunk0, BLK)],
                              eids_v.at[slot], sem).wait()
        pltpu.make_async_copy(eT.at[:, pl.ds(chunk0 * CH, BLK * CH)],
                              efeat_v.at[slot], sem).wait()

    def e_block(slot, nchunks):
        bid0 = eids_v[slot, 0, pl.ds(0, 16)][0]
        bidL = eids_v[slot, nchunks - 1, pl.ds(CH - 16, 16)][15]

        @pl.when(bid0 == bidL)
        def _():
            def g4(i, accs):
                for u in range(4):
                    accs = tuple(
                        accs[r] + efeat_v[slot, r, pl.ds(i * 64 + u * 16, 16)]
                        for r in range(16))
                return accs

            accs = lax.fori_loop(0, nchunks * 2, g4, (zvec,) * 16)
            for r in range(16):
                acc_e3[bid0, pl.ds(r * 16, 16)] += accs[r]

        @pl.when(bid0 != bidL)
        def _():
            def grp(g, c):
                cur, accs = c[0], c[1:]
                j = g // 8
                col = (g - j * 8) * 16
                idvec = eids_v[slot, j, pl.ds(col, 16)]
                gid0 = idvec[0]
                gidL = idvec[15]
                gcol = g * 16
                uniform = gid0 == gidL
                is_new = jnp.logical_and(uniform, gid0 != cur)
                vs = [efeat_v[slot, r, pl.ds(gcol, 16)] for r in range(16)]

                @pl.when(jnp.logical_and(is_new, cur >= 0))
                def _():
                    for r in range(16):
                        acc_e3[cur, pl.ds(r * 16, 16)] += accs[r]

                @pl.when(jnp.logical_not(uniform))
                def _():
                    for r in range(16):
                        plsc.addupdate_scatter(acc_e3,
                                               [idvec, lane + r * 16], vs[r])

                accs = tuple(
                    jnp.where(uniform,
                              jnp.where(is_new, zvec, accs[r]) + vs[r],
                              accs[r])
                    for r in range(16))
                cur = jnp.where(uniform, gid0, cur)
                return (cur,) + accs

            c = lax.fori_loop(0, nchunks * 8, grp,
                              (jnp.int32(-1),) + (zvec,) * 16)
            e_flush(c[0], c[1:])

    @pl.when(w == NW - 1)
    def _():
        chunk0 = NBLK_E * BLK
        pltpu.sync_copy(eids2d.at[pl.ds(chunk0, NTAIL_CH_E)],
                        eids_v.at[0, pl.ds(0, NTAIL_CH_E)])
        pltpu.sync_copy(eT.at[:, pl.ds(chunk0 * CH, NTAIL_CH_E * CH)],
                        efeat_v.at[0, :, pl.ds(0, NTAIL_CH_E * CH)])
        e_block(0, NTAIL_CH_E)

    nb0 = w * NB_N + jnp.minimum(w, EX_N)
    nbcnt = NB_N + jnp.where(w < EX_N, 1, 0)
    pltpu.sync_copy(nids2d.at[pl.ds(nb0 * BLK, 32)], nids_all)

    def n_flush(cur, accs):
        @pl.when(cur >= 0)
        def _():
            for k in range(8):
                acc_n[cur, pl.ds(k * 16, 16)] += accs[k]

    def n_stage(h, slot, sem):
        row0 = nb0 * BLK * CH + h * HCH
        pltpu.async_copy(nfeat.at[pl.ds(row0, HCH)], nfeat_v.at[slot], sem)

    def n_wait(h, slot, sem):
        row0 = nb0 * BLK * CH + h * HCH
        pltpu.make_async_copy(nfeat.at[pl.ds(row0, HCH)],
                              nfeat_v.at[slot], sem).wait()

    def n_half(slot, idrow, roff):
        hid0 = nids_all[idrow, pl.ds(roff, 16)][0]
        hidL = nids_all[idrow, pl.ds(roff + HCH - 16, 16)][15]

        @pl.when(hid0 == hidL)
        def _():
            def r4(i, accs):
                for u in range(4):
                    accs = tuple(accs[k] + nfeat_v[slot, 4 * i + u,
                                                   pl.ds(k * 16, 16)]
                                 for k in range(8))
                return accs

            accs = lax.fori_loop(0, HCH // 4, r4, (zvec,) * 8)
            for k in range(8):
                acc_n[hid0, pl.ds(k * 16, 16)] += accs[k]

        @pl.when(hid0 != hidL)
        def _():
            def grp(gr, c):
                cur, accs = c[0], c[1:]
                r0 = gr * 16
                nidvec = nids_all[idrow, pl.ds(roff + r0, 16)]
                gid0 = nidvec[0]
                gidL = nidvec[15]
                uniform = gid0 == gidL
                is_new = jnp.logical_and(uniform, gid0 != cur)

                @pl.when(jnp.logical_and(is_new, cur >= 0))
                def _():
                    for k in range(8):
                        acc_n[cur, pl.ds(k * 16, 16)] += accs[k]

                @pl.when(jnp.logical_not(uniform))
                def _():
                    for rr in range(16):
                        rid = jnp.broadcast_to(nidvec[rr], (16,))
                        for k in range(8):
                            plsc.addupdate_scatter(
                                acc_n, [rid, lane + k * 16],
                                nfeat_v[slot, r0 + rr, pl.ds(k * 16, 16)])

                tmp = tuple(jnp.where(is_new, zvec, accs[k])
                            for k in range(8))
                for rr in range(16):
                    tmp = tuple(tmp[k] + nfeat_v[slot, r0 + rr,
                                                 pl.ds(k * 16, 16)]
                                for k in range(8))
                accs = tuple(jnp.where(uniform, tmp[k], accs[k])
                             for k in range(8))
                cur = jnp.where(uniform, gid0, cur)
                return (cur,) + accs

            c = lax.fori_loop(0, 4, grp, (jnp.int32(-1),) + (zvec,) * 8)
            n_flush(c[0], c[1:])

    nhalves = nbcnt * (BLK * CH // HCH)

    e_stage(0, 0, sem_a)
    e_stage(1, 1, sem_b)
    n_stage(0, 0, sem_c)
    n_stage(1, 1, sem_d)

    def merged_pair(i, _):
        b0 = 2 * i
        b1 = b0 + 1
        e_wait(b0, 0, sem_a)

        @pl.when(b0 + 2 < ebcnt)
        def _():
            e_stage(b0 + 2, 0, sem_a)

        n_wait(b0, 0, sem_c)

        @pl.when(b0 + 2 < nhalves)
        def _():
            n_stage(b0 + 2, 0, sem_c)

        e_block(0, BLK)
        n_half(0, i, 0)
        e_wait(b1, 1, sem_b)

        @pl.when(b1 + 2 < ebcnt)
        def _():
            e_stage(b1 + 2, 1, sem_b)

        n_wait(b1, 1, sem_d)

        @pl.when(b1 + 2 < nhalves)
        def _():
            n_stage(b1 + 2, 1, sem_d)

        e_block(1, BLK)
        n_half(1, i, HCH)
        return 0

    lax.fori_loop(0, 24, merged_pair, 0)

    @pl.when(ebcnt % 2 == 1)
    def _():
        e_wait(ebcnt - 1, 0, sem_a)
        e_block(0, BLK)

    def n_pair2(j, _):
        h0 = 48 + 2 * j
        h1 = h0 + 1
        n_wait(h0, 0, sem_c)

        @pl.when(h0 + 2 < nhalves)
        def _():
            n_stage(h0 + 2, 0, sem_c)

        n_wait(h1, 1, sem_d)

        @pl.when(h1 + 2 < nhalves)
        def _():
            n_stage(h1 + 2, 1, sem_d)

        n_half(0, h0 // 2, 0)
        n_half(1, h1 // 2, HCH)
        return 0

    lax.fori_loop(0, (nhalves - 48) // 2, n_pair2, 0)

    @pl.when(w == NW - 2)
    def _():
        pltpu.sync_copy(nids2d.at[pl.ds(NBLK_N * BLK, BLK)],
                        nids_all.at[pl.ds(0, BLK)])

        def one(j, _):
            row0 = NBLK_N * BLK * CH + j * HCH
            pltpu.sync_copy(nfeat.at[pl.ds(row0, HCH)], nfeat_v.at[0])
            n_half(0, j // 2, (j % 2) * HCH)
            return 0

        lax.fori_loop(0, NTAIL_CH_N * 2, one, 0)

    @pl.when(w == NW - 1)
    def _():
        pltpu.sync_copy(ntail_ids, ntail_ids_v)
        pltpu.sync_copy(nfeat.at[pl.ds(N_CHUNKS_N * CH, N_TAIL)],
                        nfeat_v.at[0, pl.ds(0, N_TAIL)])
        for g in range(N_TAIL // 16):
            idvec = ntail_ids_v[pl.ds(g * 16, 16)]
            for rr in range(16):
                rid = jnp.broadcast_to(idvec[rr], (16,))
                for k in range(8):
                    plsc.addupdate_scatter(
                        acc_n, [rid, lane + k * 16],
                        nfeat_v[0, g * 16 + rr, pl.ds(k * 16, 16)])

    pltpu.sync_copy(acc_n, out_n.at[w])
    pltpu.sync_copy(acc_e3, out_e3.at[w])


@functools.partial(
    pl.kernel,
    out_type=(
        jax.ShapeDtypeStruct((NW, NUM_GRAPHS, D_NODE), jnp.float32),
        jax.ShapeDtypeStruct((NW, NUM_GRAPHS, 16 * D_EDGE), jnp.float32),
    ),
    mesh=plsc.VectorSubcoreMesh(core_axis_name="c", subcore_axis_name="s"),
    scratch_types=[
        pltpu.VMEM((2, BLK, CH), jnp.int32),
        pltpu.VMEM((2, D_EDGE, BLK * CH), jnp.float32),
        pltpu.VMEM((32, CH), jnp.int32),
        pltpu.VMEM((2, HCH, D_NODE), jnp.float32),
        pltpu.VMEM((N_TAIL,), jnp.int32),
        pltpu.VMEM((NUM_GRAPHS, D_NODE), jnp.float32),
        pltpu.VMEM((NUM_GRAPHS, 16 * D_EDGE), jnp.float32),
        pltpu.SemaphoreType.DMA,
        pltpu.SemaphoreType.DMA,
        pltpu.SemaphoreType.DMA,
        pltpu.SemaphoreType.DMA,
    ],
    compiler_params=pltpu.CompilerParams(needs_layout_passes=False),
)
def _sc_segsums(nfeat, eT, nids2d, ntail_ids, eids2d, out_n, out_e3,
                *scratch):
    _sc_body(nfeat, eT, nids2d, ntail_ids, eids2d, out_n, out_e3, *scratch)


def _final_body(pn_ref, pe_ref, fold_ref, g_ref, wn_ref, we_ref, wg_ref,
                b_ref, out_ref):
    agg_n = jnp.sum(pn_ref[...], axis=0)
    pe = jnp.sum(pe_ref[...], axis=0)
    agg_e = jax.lax.dot_general(
        pe, fold_ref[...], (((1,), (0,)), ((), ())),
        precision=jax.lax.Precision.HIGHEST,
        preferred_element_type=jnp.float32)
    acc = jax.lax.dot_general(
        agg_n, wn_ref[...], (((1,), (1,)), ((), ())),
        preferred_element_type=jnp.float32)
    acc += jax.lax.dot_general(
        agg_e, we_ref[...], (((1,), (1,)), ((), ())),
        preferred_element_type=jnp.float32)
    acc += jax.lax.dot_general(
        g_ref[...], wg_ref[...], (((1,), (1,)), ((), ())),
        preferred_element_type=jnp.float32)
    out_ref[...] = acc + b_ref[...]


def kernel(node_features, edge_features, global_features, node_graph_ids,
           edge_graph_ids, W_node, W_edges, W_global, bias):
    nids2d = jnp.pad(
        node_graph_ids[:N_CHUNKS_N * CH],
        (0, N_IDROWS_N * CH - N_CHUNKS_N * CH)).reshape(N_IDROWS_N, CH)
    ntail_ids = node_graph_ids[N_CHUNKS_N * CH:]
    eids2d = edge_graph_ids.reshape(N_CHUNKS_E, CH)
    part_n, part_e3 = _sc_segsums(node_features, edge_features.T, nids2d,
                                  ntail_ids, eids2d)
    fold = (jnp.arange(16 * D_EDGE)[:, None] // 16
            == jnp.arange(D_EDGE)[None, :]).astype(jnp.float32)
    return pl.pallas_call(
        _final_body,
        out_shape=jax.ShapeDtypeStruct((NUM_GRAPHS, D_OUT), jnp.float32),
    )(part_n, part_e3, fold, global_features, W_node, W_edges, W_global,
      bias.reshape(1, D_OUT))
